# baseline (speedup 1.0000x reference)
import functools

import jax
import jax.numpy as jnp
from jax import lax
from jax.experimental import pallas as pl
from jax.experimental.pallas import tpu as pltpu
from jax.experimental.pallas import tpu_sc as plsc

N = 10000
E = 160000
DF = 256
DH = 256
DC = 64

NCORE = 2
NSUB = 16
LANES = 16

EB = 128
E_PAD = 163840
DUMP = N
N_ACC = 10112
ZROWS = 40
CH = 200
NCH = N // CH

NB = 1000

_mesh = lambda: plsc.VectorSubcoreMesh(core_axis_name="c", subcore_axis_name="s")


DW = 128


RING = 2
NHALF = 2


def _chunk_loop(sid, body):
    @pl.loop(0, pl.cdiv(NCH, NSUB))
    def _(k):
        c = sid + k * NSUB

        @pl.when(c < NCH)
        def _():
            body(c)


def _edge_phase(HB, hp_hbm, acc_sh, srcb_v, dstb_v, bufs, sems):
    def start(lb, j):
        pltpu.async_copy(hp_hbm.at[srcb_v.at[lb]], bufs[j], sems[j])

    def wait_g(lb, j):
        pltpu.make_async_copy(hp_hbm.at[srcb_v.at[lb]], bufs[j], sems[j]).wait()

    for j in range(RING):
        start(j, j)

    @pl.loop(0, HB // RING)
    def _(k):
        for j in range(RING):
            lb = k * RING + j
            wait_g(lb, j)
            pltpu.sync_copy(bufs[j], acc_sh.at[dstb_v.at[lb]], add=True)
            nb = lb + RING

            @pl.when(nb < HB)
            def _():
                start(nb, j)


def _deg_kernel(dstr_hbm, ones_hbm, z_hbm, out_hbm, dstb_v, ones_v, acc_sh):
    cid = lax.axis_index("c")
    sid = lax.axis_index("s")

    wid = cid * NSUB + sid
    nbatch = E_PAD // (NCORE * NSUB * EB)
    pltpu.sync_copy(dstr_hbm.at[pl.ds(wid * nbatch, nbatch)], dstb_v)
    pltpu.sync_copy(ones_hbm, ones_v)

    _chunk_loop(sid, lambda c: pltpu.sync_copy(
        z_hbm, acc_sh.at[pl.ds(c * CH, CH)]))

    plsc.subcore_barrier()

    @pl.loop(0, nbatch)
    def _(b):
        pltpu.sync_copy(ones_v, acc_sh.at[dstb_v.at[b]], add=True)

    plsc.subcore_barrier()

    _chunk_loop(sid, lambda c: pltpu.sync_copy(
        acc_sh.at[pl.ds(c * CH, CH)],
        out_hbm.at[pl.ds(cid * N + c * CH, CH)]))


def _deg_counts(dstr):
    kern = pl.kernel(
        _deg_kernel,
        out_type=jax.ShapeDtypeStruct((NCORE * N, DW), jnp.float32),
        mesh=_mesh(),
        scratch_types=[
            pltpu.VMEM((E_PAD // (NCORE * NSUB * EB), EB), jnp.int32),
            pltpu.VMEM((EB, DW), jnp.float32),
            pltpu.VMEM_SHARED((N_ACC, DW), jnp.float32),
        ],
    )
    return kern(dstr,
                jnp.ones((EB, DW), jnp.float32),
                jnp.zeros((CH, DW), jnp.float32))


def _spmm_kernel(hp_hbm, src2r_hbm, dstr_hbm, out_hbm,
                 srcb_v, dstb_v, b0_v, b1_v, acc_sh, s0, s1):
    cid = lax.axis_index("c")
    sid = lax.axis_index("s")

    nbatch = E_PAD // (NSUB * EB)
    HB = nbatch // NHALF

    _chunk_loop(sid, lambda c: pltpu.sync_copy(
        hp_hbm.at[pl.ds(cid * N + c * CH, CH)],
        acc_sh.at[pl.ds(c * CH, CH)]))

    plsc.subcore_barrier()

    for h in range(NHALF):
        pltpu.sync_copy(
            src2r_hbm.at[pl.ds((cid * NSUB + sid) * nbatch + h * HB, HB)],
            srcb_v)
        pltpu.sync_copy(
            dstr_hbm.at[pl.ds(sid * nbatch + h * HB, HB)], dstb_v)
        _edge_phase(HB, hp_hbm, acc_sh, srcb_v, dstb_v,
                    (b0_v, b1_v), (s0, s1))

    plsc.subcore_barrier()

    _chunk_loop(sid, lambda c: pltpu.sync_copy(
        acc_sh.at[pl.ds(c * CH, CH)],
        out_hbm.at[pl.ds(cid * N + c * CH, CH)]))


def _spmm(hp2, src2r, dstr):
    hb = E_PAD // (NSUB * EB * NHALF)
    kern = pl.kernel(
        _spmm_kernel,
        out_type=jax.ShapeDtypeStruct((NCORE * N, DW), jnp.float32),
        mesh=_mesh(),
        scratch_types=[
            pltpu.VMEM((hb, EB), jnp.int32),
            pltpu.VMEM((hb, EB), jnp.int32),
            pltpu.VMEM((EB, DW), jnp.float32),
            pltpu.VMEM((EB, DW), jnp.float32),
            pltpu.VMEM_SHARED((N_ACC, DW), jnp.float32),
            pltpu.SemaphoreType.DMA,
            pltpu.SemaphoreType.DMA,
        ],
    )
    return kern(hp2, src2r, dstr)


def _spmm2_kernel(hp_hbm, srcr_hbm, dstr_hbm, out_hbm,
                  srcb_v, dstb_v, b0_v, b1_v, acc_sh, s0, s1):
    cid = lax.axis_index("c")
    sid = lax.axis_index("s")

    nbatch = E_PAD // (NCORE * NSUB * EB)
    base = (cid * NSUB + sid) * nbatch

    _chunk_loop(sid, lambda c: pltpu.sync_copy(
        hp_hbm.at[pl.ds(c * CH, CH)],
        acc_sh.at[pl.ds(c * CH, CH)]))

    plsc.subcore_barrier()

    pltpu.sync_copy(srcr_hbm.at[pl.ds(base, nbatch)], srcb_v)
    pltpu.sync_copy(dstr_hbm.at[pl.ds(base, nbatch)], dstb_v)
    _edge_phase(nbatch, hp_hbm, acc_sh, srcb_v, dstb_v,
                (b0_v, b1_v), (s0, s1))

    plsc.subcore_barrier()

    _chunk_loop(sid, lambda c: pltpu.sync_copy(
        acc_sh.at[pl.ds(c * CH, CH)],
        out_hbm.at[pl.ds(cid * N + c * CH, CH)]))


def _spmm_l2(hp2p, srcr, dstr):
    hb = E_PAD // (NCORE * NSUB * EB)
    kern = pl.kernel(
        _spmm2_kernel,
        out_type=jax.ShapeDtypeStruct((NCORE * N, DW), jnp.float32),
        mesh=_mesh(),
        scratch_types=[
            pltpu.VMEM((hb, EB), jnp.int32),
            pltpu.VMEM((hb, EB), jnp.int32),
            pltpu.VMEM((EB, DW), jnp.float32),
            pltpu.VMEM((EB, DW), jnp.float32),
            pltpu.VMEM_SHARED((N_ACC, DW), jnp.float32),
            pltpu.SemaphoreType.DMA,
            pltpu.SemaphoreType.DMA,
        ],
    )
    return kern(hp2p, srcr, dstr)


def _dis(d0, d1):
    return lax.rsqrt(d0[:, 0] + d1[:, 0] + 1.0)


def _tc1a_body(x_ref, w_ref, out_ref):
    out_ref[...] = jnp.dot(x_ref[...], w_ref[...],
                           preferred_element_type=jnp.float32,
                           precision=lax.Precision.HIGHEST)


def _tc1a(x, W1):
    f = pl.pallas_call(
        _tc1a_body,
        grid=(N // NB,),
        in_specs=[
            pl.BlockSpec((NB, DF), lambda i: (i, 0)),
            pl.BlockSpec((DF, DH), lambda i: (0, 0)),
        ],
        out_specs=pl.BlockSpec((NB, DH), lambda i: (i, 0)),
        out_shape=jax.ShapeDtypeStruct((N, DH), jnp.float32),
    )
    return f(x, W1)


def _tc1b_body(h_ref, d0_ref, d1_ref, out_ref):
    dis = _dis(d0_ref[...], d1_ref[...])
    h = h_ref[...]
    out_ref[0] = dis[:, None] * h[:, :DH // 2]
    out_ref[1] = dis[:, None] * h[:, DH // 2:]


def _tc1b(h1, degp):
    f = pl.pallas_call(
        _tc1b_body,
        grid=(N // NB,),
        in_specs=[
            pl.BlockSpec((NB, DH), lambda i: (i, 0)),
            pl.BlockSpec((NB, DW), lambda i: (i, 0)),
            pl.BlockSpec((NB, DW), lambda i: (N // NB + i, 0)),
        ],
        out_specs=pl.BlockSpec((2, NB, DH // 2), lambda i: (0, i, 0)),
        out_shape=jax.ShapeDtypeStruct((2, N, DH // 2), jnp.float32),
    )
    return f(h1, degp, degp).reshape(2 * N, DH // 2)


def _tc2_body(a0_ref, a1_ref, d0_ref, d1_ref, w_ref, b_ref, out_ref):
    dis = _dis(d0_ref[...], d1_ref[...])
    z0 = jax.nn.relu(dis[:, None] * a0_ref[...] + b_ref[0][None, :])
    z1 = jax.nn.relu(dis[:, None] * a1_ref[...] + b_ref[1][None, :])
    h2 = (jnp.dot(z0, w_ref[:DH // 2],
                  preferred_element_type=jnp.float32,
                  precision=lax.Precision.HIGHEST)
          + jnp.dot(z1, w_ref[DH // 2:],
                    preferred_element_type=jnp.float32,
                    precision=lax.Precision.HIGHEST))
    out_ref[...] = jnp.concatenate(
        [dis[:, None] * h2, jnp.zeros((NB, DC), jnp.float32)], axis=1)


def _tc2(aggr1, degp, W2, b1):
    f = pl.pallas_call(
        _tc2_body,
        grid=(N // NB,),
        in_specs=[
            pl.BlockSpec((NB, DH // 2), lambda i: (i, 0)),
            pl.BlockSpec((NB, DH // 2), lambda i: (N // NB + i, 0)),
            pl.BlockSpec((NB, DW), lambda i: (i, 0)),
            pl.BlockSpec((NB, DW), lambda i: (N // NB + i, 0)),
            pl.BlockSpec((DH, DC), lambda i: (0, 0)),
            pl.BlockSpec((2, DH // 2), lambda i: (0, 0)),
        ],
        out_specs=pl.BlockSpec((NB, 2 * DC), lambda i: (i, 0)),
        out_shape=jax.ShapeDtypeStruct((N, 2 * DC), jnp.float32),
    )
    return f(aggr1, aggr1, degp, degp, W2, b1.reshape(2, DH // 2))


def _tc3_body(p0_ref, p1_ref, hp_ref, d0_ref, d1_ref, b_ref, out_ref):
    dis = _dis(d0_ref[...], d1_ref[...])
    aggr = (p0_ref[...] + p1_ref[...] - hp_ref[...])[:, :DC]
    logits = dis[:, None] * aggr + b_ref[...]
    m = jnp.max(logits, axis=1, keepdims=True)
    lse = m + jnp.log(jnp.sum(jnp.exp(logits - m), axis=1, keepdims=True))
    out_ref[...] = logits - lse


def _tc3(parts2, hp2p, degp, b2):
    f = pl.pallas_call(
        _tc3_body,
        grid=(N // NB,),
        in_specs=[
            pl.BlockSpec((NB, 2 * DC), lambda i: (i, 0)),
            pl.BlockSpec((NB, 2 * DC), lambda i: (N // NB + i, 0)),
            pl.BlockSpec((NB, 2 * DC), lambda i: (i, 0)),
            pl.BlockSpec((NB, DW), lambda i: (i, 0)),
            pl.BlockSpec((NB, DW), lambda i: (N // NB + i, 0)),
            pl.BlockSpec((1, DC), lambda i: (0, 0)),
        ],
        out_specs=pl.BlockSpec((NB, DC), lambda i: (i, 0)),
        out_shape=jax.ShapeDtypeStruct((N, DC), jnp.float32),
    )
    return f(parts2, parts2, hp2p, degp, degp, b2.reshape(1, DC))


def kernel(x, edge_index, W1, b1, W2, b2):
    src = edge_index[0].astype(jnp.int32)
    dst = edge_index[1].astype(jnp.int32)
    pad = E_PAD - E
    k = jnp.arange(pad, dtype=jnp.int32)
    src_p = jnp.concatenate([src, k * 97 % N])
    dst_p = jnp.concatenate([dst, DUMP + (k % (N_ACC - N))])
    src2r = jnp.concatenate([src_p, src_p + N]).reshape(-1, EB)
    srcr = src_p.reshape(-1, EB)
    dstr = dst_p.reshape(-1, EB)

    h1 = _tc1a(x, W1)
    degp = _deg_counts(dstr)
    hp1 = _tc1b(h1, degp)
    aggr1 = _spmm(hp1, src2r, dstr)
    hp2p = _tc2(aggr1, degp, W2, b1)
    parts2 = _spmm_l2(hp2p, srcr, dstr)
    return _tc3(parts2, hp2p, degp, b2)


# speedup vs baseline: 1.0203x; 1.0203x over previous
import functools

import jax
import jax.numpy as jnp
from jax import lax
from jax.experimental import pallas as pl
from jax.experimental.pallas import tpu as pltpu
from jax.experimental.pallas import tpu_sc as plsc

N = 10000
E = 160000
DF = 256
DH = 256
DC = 64

NCORE = 2
NSUB = 16
LANES = 16

EB = 128
E_PAD = 163840
DUMP = N
N_ACC = 10112
ZROWS = 40
CH = 200
NCH = N // CH

NB = 1000

_mesh = lambda: plsc.VectorSubcoreMesh(core_axis_name="c", subcore_axis_name="s")


DW = 128


RING = 2
NHALF = 4
EB1 = 64


def _chunk_loop(sid, body):
    @pl.loop(0, pl.cdiv(NCH, NSUB))
    def _(k):
        c = sid + k * NSUB

        @pl.when(c < NCH)
        def _():
            body(c)


def _edge_phase(HB, hp_hbm, acc_sh, srcb_v, dstb_v, bufs, sems):
    ring = len(bufs)

    def start(lb, j):
        pltpu.async_copy(hp_hbm.at[srcb_v.at[lb]], bufs[j], sems[j])

    def wait_g(lb, j):
        pltpu.make_async_copy(hp_hbm.at[srcb_v.at[lb]], bufs[j], sems[j]).wait()

    for j in range(ring):
        start(j, j)

    @pl.loop(0, HB // ring)
    def _(k):
        for j in range(ring):
            lb = k * ring + j
            wait_g(lb, j)
            pltpu.sync_copy(bufs[j], acc_sh.at[dstb_v.at[lb]], add=True)
            nb = lb + ring

            @pl.when(nb < HB)
            def _():
                start(nb, j)


def _deg_kernel(dstr_hbm, ones_hbm, z_hbm, out_hbm, dstb_v, ones_v, acc_sh):
    cid = lax.axis_index("c")
    sid = lax.axis_index("s")

    wid = cid * NSUB + sid
    nbatch = E_PAD // (NCORE * NSUB * EB)
    pltpu.sync_copy(dstr_hbm.at[pl.ds(wid * nbatch, nbatch)], dstb_v)
    pltpu.sync_copy(ones_hbm, ones_v)

    _chunk_loop(sid, lambda c: pltpu.sync_copy(
        z_hbm, acc_sh.at[pl.ds(c * CH, CH)]))

    plsc.subcore_barrier()

    @pl.loop(0, nbatch)
    def _(b):
        pltpu.sync_copy(ones_v, acc_sh.at[dstb_v.at[b]], add=True)

    plsc.subcore_barrier()

    _chunk_loop(sid, lambda c: pltpu.sync_copy(
        acc_sh.at[pl.ds(c * CH, CH)],
        out_hbm.at[pl.ds(cid * N + c * CH, CH)]))


def _deg_counts(dstr):
    kern = pl.kernel(
        _deg_kernel,
        out_type=jax.ShapeDtypeStruct((NCORE * N, DW), jnp.float32),
        mesh=_mesh(),
        scratch_types=[
            pltpu.VMEM((E_PAD // (NCORE * NSUB * EB), EB), jnp.int32),
            pltpu.VMEM((EB, DW), jnp.float32),
            pltpu.VMEM_SHARED((N_ACC, DW), jnp.float32),
        ],
    )
    return kern(dstr,
                jnp.ones((EB, DW), jnp.float32),
                jnp.zeros((CH, DW), jnp.float32))


def _spmm_kernel(hp_hbm, src2r_hbm, dstr_hbm, out_hbm,
                 srcb_v, dstb_v, b0_v, b1_v, b2_v, b3_v,
                 acc_sh, s0, s1, s2, s3):
    cid = lax.axis_index("c")
    sid = lax.axis_index("s")

    nbatch = E_PAD // (NSUB * EB1)
    HB = nbatch // NHALF

    _chunk_loop(sid, lambda c: pltpu.sync_copy(
        hp_hbm.at[pl.ds(cid * N + c * CH, CH)],
        acc_sh.at[pl.ds(c * CH, CH)]))

    plsc.subcore_barrier()

    for h in range(NHALF):
        pltpu.sync_copy(
            src2r_hbm.at[pl.ds((cid * NSUB + sid) * nbatch + h * HB, HB)],
            srcb_v)
        pltpu.sync_copy(
            dstr_hbm.at[pl.ds(sid * nbatch + h * HB, HB)], dstb_v)
        _edge_phase(HB, hp_hbm, acc_sh, srcb_v, dstb_v,
                    (b0_v, b1_v, b2_v, b3_v), (s0, s1, s2, s3))

    plsc.subcore_barrier()

    _chunk_loop(sid, lambda c: pltpu.sync_copy(
        acc_sh.at[pl.ds(c * CH, CH)],
        out_hbm.at[pl.ds(cid * N + c * CH, CH)]))


def _spmm(hp2, src2r, dstr):
    hb = E_PAD // (NSUB * EB1 * NHALF)
    kern = pl.kernel(
        _spmm_kernel,
        out_type=jax.ShapeDtypeStruct((NCORE * N, DW), jnp.float32),
        mesh=_mesh(),
        scratch_types=[
            pltpu.VMEM((hb, EB1), jnp.int32),
            pltpu.VMEM((hb, EB1), jnp.int32),
            pltpu.VMEM((EB1, DW), jnp.float32),
            pltpu.VMEM((EB1, DW), jnp.float32),
            pltpu.VMEM((EB1, DW), jnp.float32),
            pltpu.VMEM((EB1, DW), jnp.float32),
            pltpu.VMEM_SHARED((N_ACC, DW), jnp.float32),
            pltpu.SemaphoreType.DMA,
            pltpu.SemaphoreType.DMA,
            pltpu.SemaphoreType.DMA,
            pltpu.SemaphoreType.DMA,
        ],
    )
    return kern(hp2, src2r, dstr)


def _spmm2_kernel(hp_hbm, srcr_hbm, dstr_hbm, out_hbm,
                  srcb_v, dstb_v, b0_v, b1_v, acc_sh, s0, s1):
    cid = lax.axis_index("c")
    sid = lax.axis_index("s")

    nbatch = E_PAD // (NCORE * NSUB * EB)
    base = (cid * NSUB + sid) * nbatch

    _chunk_loop(sid, lambda c: pltpu.sync_copy(
        hp_hbm.at[pl.ds(c * CH, CH)],
        acc_sh.at[pl.ds(c * CH, CH)]))

    plsc.subcore_barrier()

    pltpu.sync_copy(srcr_hbm.at[pl.ds(base, nbatch)], srcb_v)
    pltpu.sync_copy(dstr_hbm.at[pl.ds(base, nbatch)], dstb_v)
    _edge_phase(nbatch, hp_hbm, acc_sh, srcb_v, dstb_v,
                (b0_v, b1_v), (s0, s1))

    plsc.subcore_barrier()

    _chunk_loop(sid, lambda c: pltpu.sync_copy(
        acc_sh.at[pl.ds(c * CH, CH)],
        out_hbm.at[pl.ds(cid * N + c * CH, CH)]))


def _spmm_l2(hp2p, srcr, dstr):
    hb = E_PAD // (NCORE * NSUB * EB)
    kern = pl.kernel(
        _spmm2_kernel,
        out_type=jax.ShapeDtypeStruct((NCORE * N, DW), jnp.float32),
        mesh=_mesh(),
        scratch_types=[
            pltpu.VMEM((hb, EB), jnp.int32),
            pltpu.VMEM((hb, EB), jnp.int32),
            pltpu.VMEM((EB, DW), jnp.float32),
            pltpu.VMEM((EB, DW), jnp.float32),
            pltpu.VMEM_SHARED((N_ACC, DW), jnp.float32),
            pltpu.SemaphoreType.DMA,
            pltpu.SemaphoreType.DMA,
        ],
    )
    return kern(hp2p, srcr, dstr)


def _dis(d0, d1):
    return lax.rsqrt(d0[:, 0] + d1[:, 0] + 1.0)


def _tc1a_body(x_ref, w_ref, out_ref):
    out_ref[...] = jnp.dot(x_ref[...], w_ref[...],
                           preferred_element_type=jnp.float32,
                           precision=lax.Precision.HIGHEST)


def _tc1a(x, W1):
    f = pl.pallas_call(
        _tc1a_body,
        grid=(N // NB,),
        in_specs=[
            pl.BlockSpec((NB, DF), lambda i: (i, 0)),
            pl.BlockSpec((DF, DH), lambda i: (0, 0)),
        ],
        out_specs=pl.BlockSpec((NB, DH), lambda i: (i, 0)),
        out_shape=jax.ShapeDtypeStruct((N, DH), jnp.float32),
    )
    return f(x, W1)


def _tc1b_body(h_ref, d0_ref, d1_ref, out_ref):
    dis = _dis(d0_ref[...], d1_ref[...])
    h = h_ref[...]
    out_ref[0] = dis[:, None] * h[:, :DH // 2]
    out_ref[1] = dis[:, None] * h[:, DH // 2:]


def _tc1b(h1, degp):
    f = pl.pallas_call(
        _tc1b_body,
        grid=(N // NB,),
        in_specs=[
            pl.BlockSpec((NB, DH), lambda i: (i, 0)),
            pl.BlockSpec((NB, DW), lambda i: (i, 0)),
            pl.BlockSpec((NB, DW), lambda i: (N // NB + i, 0)),
        ],
        out_specs=pl.BlockSpec((2, NB, DH // 2), lambda i: (0, i, 0)),
        out_shape=jax.ShapeDtypeStruct((2, N, DH // 2), jnp.float32),
    )
    return f(h1, degp, degp).reshape(2 * N, DH // 2)


def _tc2_body(a0_ref, a1_ref, d0_ref, d1_ref, w_ref, b_ref, out_ref):
    dis = _dis(d0_ref[...], d1_ref[...])
    z0 = jax.nn.relu(dis[:, None] * a0_ref[...] + b_ref[0][None, :])
    z1 = jax.nn.relu(dis[:, None] * a1_ref[...] + b_ref[1][None, :])
    h2 = (jnp.dot(z0, w_ref[:DH // 2],
                  preferred_element_type=jnp.float32,
                  precision=lax.Precision.HIGHEST)
          + jnp.dot(z1, w_ref[DH // 2:],
                    preferred_element_type=jnp.float32,
                    precision=lax.Precision.HIGHEST))
    out_ref[...] = jnp.concatenate(
        [dis[:, None] * h2, jnp.zeros((NB, DC), jnp.float32)], axis=1)


def _tc2(aggr1, degp, W2, b1):
    f = pl.pallas_call(
        _tc2_body,
        grid=(N // NB,),
        in_specs=[
            pl.BlockSpec((NB, DH // 2), lambda i: (i, 0)),
            pl.BlockSpec((NB, DH // 2), lambda i: (N // NB + i, 0)),
            pl.BlockSpec((NB, DW), lambda i: (i, 0)),
            pl.BlockSpec((NB, DW), lambda i: (N // NB + i, 0)),
            pl.BlockSpec((DH, DC), lambda i: (0, 0)),
            pl.BlockSpec((2, DH // 2), lambda i: (0, 0)),
        ],
        out_specs=pl.BlockSpec((NB, 2 * DC), lambda i: (i, 0)),
        out_shape=jax.ShapeDtypeStruct((N, 2 * DC), jnp.float32),
    )
    return f(aggr1, aggr1, degp, degp, W2, b1.reshape(2, DH // 2))


def _tc3_body(p0_ref, p1_ref, hp_ref, d0_ref, d1_ref, b_ref, out_ref):
    dis = _dis(d0_ref[...], d1_ref[...])
    aggr = (p0_ref[...] + p1_ref[...] - hp_ref[...])[:, :DC]
    logits = dis[:, None] * aggr + b_ref[...]
    m = jnp.max(logits, axis=1, keepdims=True)
    lse = m + jnp.log(jnp.sum(jnp.exp(logits - m), axis=1, keepdims=True))
    out_ref[...] = logits - lse


def _tc3(parts2, hp2p, degp, b2):
    f = pl.pallas_call(
        _tc3_body,
        grid=(N // NB,),
        in_specs=[
            pl.BlockSpec((NB, 2 * DC), lambda i: (i, 0)),
            pl.BlockSpec((NB, 2 * DC), lambda i: (N // NB + i, 0)),
            pl.BlockSpec((NB, 2 * DC), lambda i: (i, 0)),
            pl.BlockSpec((NB, DW), lambda i: (i, 0)),
            pl.BlockSpec((NB, DW), lambda i: (N // NB + i, 0)),
            pl.BlockSpec((1, DC), lambda i: (0, 0)),
        ],
        out_specs=pl.BlockSpec((NB, DC), lambda i: (i, 0)),
        out_shape=jax.ShapeDtypeStruct((N, DC), jnp.float32),
    )
    return f(parts2, parts2, hp2p, degp, degp, b2.reshape(1, DC))


def kernel(x, edge_index, W1, b1, W2, b2):
    src = edge_index[0].astype(jnp.int32)
    dst = edge_index[1].astype(jnp.int32)
    pad = E_PAD - E
    k = jnp.arange(pad, dtype=jnp.int32)
    src_p = jnp.concatenate([src, k * 97 % N])
    dst_p = jnp.concatenate([dst, DUMP + (k % (N_ACC - N))])
    src2r = jnp.concatenate([src_p, src_p + N]).reshape(-1, EB1)
    dstr1 = dst_p.reshape(-1, EB1)
    srcr = src_p.reshape(-1, EB)
    dstr = dst_p.reshape(-1, EB)

    h1 = _tc1a(x, W1)
    degp = _deg_counts(dstr)
    hp1 = _tc1b(h1, degp)
    aggr1 = _spmm(hp1, src2r, dstr1)
    hp2p = _tc2(aggr1, degp, W2, b1)
    parts2 = _spmm_l2(hp2p, srcr, dstr)
    return _tc3(parts2, hp2p, degp, b2)


# speedup vs baseline: 1.0452x; 1.0244x over previous
import functools

import jax
import jax.numpy as jnp
from jax import lax
from jax.experimental import pallas as pl
from jax.experimental.pallas import tpu as pltpu
from jax.experimental.pallas import tpu_sc as plsc

N = 10000
E = 160000
DF = 256
DH = 256
DC = 64

NCORE = 2
NSUB = 16
LANES = 16

EB = 128
E_PAD = 163840
DUMP = N
N_ACC = 10112
ZROWS = 40
CH = 200
NCH = N // CH

NB = 1000

_mesh = lambda: plsc.VectorSubcoreMesh(core_axis_name="c", subcore_axis_name="s")


DW = 128


RING = 2
NHALF = 4
EB1 = 64


def _chunk_loop(sid, body):
    @pl.loop(0, pl.cdiv(NCH, NSUB))
    def _(k):
        c = sid + k * NSUB

        @pl.when(c < NCH)
        def _():
            body(c)


def _edge_phase(HB, hp_hbm, acc_sh, srcb_v, dstb_v, bufs, sems):
    ring = len(bufs)

    def start(lb, j):
        pltpu.async_copy(hp_hbm.at[srcb_v.at[lb]], bufs[j], sems[j])

    def wait_g(lb, j):
        pltpu.make_async_copy(hp_hbm.at[srcb_v.at[lb]], bufs[j], sems[j]).wait()

    for j in range(ring):
        start(j, j)

    @pl.loop(0, HB // ring)
    def _(k):
        for j in range(ring):
            lb = k * ring + j
            wait_g(lb, j)
            pltpu.sync_copy(bufs[j], acc_sh.at[dstb_v.at[lb]], add=True)
            nb = lb + ring

            @pl.when(nb < HB)
            def _():
                start(nb, j)


def _deg_kernel(dstr_hbm, ones_hbm, z_hbm, out_hbm, dstb_v, ones_v, acc_sh,
                sem):
    cid = lax.axis_index("c")
    sid = lax.axis_index("s")

    wid = cid * NSUB + sid
    nbatch = E_PAD // (NCORE * NSUB * EB)
    pltpu.sync_copy(dstr_hbm.at[pl.ds(wid * nbatch, nbatch)], dstb_v)
    pltpu.sync_copy(ones_hbm, ones_v)

    _chunk_loop(sid, lambda c: pltpu.sync_copy(
        z_hbm, acc_sh.at[pl.ds(c * CH, CH)]))

    plsc.subcore_barrier()

    @pl.loop(0, nbatch // 8)
    def _(g):
        for j in range(8):
            pltpu.async_copy(ones_v, acc_sh.at[dstb_v.at[g * 8 + j]],
                             sem, add=True)
        for j in range(8):
            pltpu.make_async_copy(ones_v, acc_sh.at[dstb_v.at[g * 8 + j]],
                                  sem).wait()

    plsc.subcore_barrier()

    _chunk_loop(sid, lambda c: pltpu.sync_copy(
        acc_sh.at[pl.ds(c * CH, CH)],
        out_hbm.at[pl.ds(cid * N + c * CH, CH)]))


def _deg_counts(dstr):
    kern = pl.kernel(
        _deg_kernel,
        out_type=jax.ShapeDtypeStruct((NCORE * N, DW), jnp.float32),
        mesh=_mesh(),
        scratch_types=[
            pltpu.VMEM((E_PAD // (NCORE * NSUB * EB), EB), jnp.int32),
            pltpu.VMEM((EB, DW), jnp.float32),
            pltpu.VMEM_SHARED((N_ACC, DW), jnp.float32),
            pltpu.SemaphoreType.DMA,
        ],
    )
    return kern(dstr,
                jnp.ones((EB, DW), jnp.float32),
                jnp.zeros((CH, DW), jnp.float32))


def _spmm_kernel(hp_hbm, src2r_hbm, dstr_hbm, out_hbm,
                 srcb_v, dstb_v, b0_v, b1_v, b2_v, b3_v,
                 acc_sh, s0, s1, s2, s3):
    cid = lax.axis_index("c")
    sid = lax.axis_index("s")

    nbatch = E_PAD // (NSUB * EB1)
    HB = nbatch // NHALF

    _chunk_loop(sid, lambda c: pltpu.sync_copy(
        hp_hbm.at[pl.ds(cid * N + c * CH, CH)],
        acc_sh.at[pl.ds(c * CH, CH)]))

    plsc.subcore_barrier()

    for h in range(NHALF):
        pltpu.sync_copy(
            src2r_hbm.at[pl.ds((cid * NSUB + sid) * nbatch + h * HB, HB)],
            srcb_v)
        pltpu.sync_copy(
            dstr_hbm.at[pl.ds(sid * nbatch + h * HB, HB)], dstb_v)
        _edge_phase(HB, hp_hbm, acc_sh, srcb_v, dstb_v,
                    (b0_v, b1_v, b2_v, b3_v), (s0, s1, s2, s3))

    plsc.subcore_barrier()

    _chunk_loop(sid, lambda c: pltpu.sync_copy(
        acc_sh.at[pl.ds(c * CH, CH)],
        out_hbm.at[pl.ds(cid * N + c * CH, CH)]))


def _spmm(hp2, src2r, dstr):
    hb = E_PAD // (NSUB * EB1 * NHALF)
    kern = pl.kernel(
        _spmm_kernel,
        out_type=jax.ShapeDtypeStruct((NCORE * N, DW), jnp.float32),
        mesh=_mesh(),
        scratch_types=[
            pltpu.VMEM((hb, EB1), jnp.int32),
            pltpu.VMEM((hb, EB1), jnp.int32),
            pltpu.VMEM((EB1, DW), jnp.float32),
            pltpu.VMEM((EB1, DW), jnp.float32),
            pltpu.VMEM((EB1, DW), jnp.float32),
            pltpu.VMEM((EB1, DW), jnp.float32),
            pltpu.VMEM_SHARED((N_ACC, DW), jnp.float32),
            pltpu.SemaphoreType.DMA,
            pltpu.SemaphoreType.DMA,
            pltpu.SemaphoreType.DMA,
            pltpu.SemaphoreType.DMA,
        ],
    )
    return kern(hp2, src2r, dstr)


def _spmm2_kernel(hp_hbm, srcr_hbm, dstr_hbm, out_hbm,
                  srcb_v, dstb_v, b0_v, b1_v, acc_sh, s0, s1):
    cid = lax.axis_index("c")
    sid = lax.axis_index("s")

    nbatch = E_PAD // (NCORE * NSUB * EB)
    base = (cid * NSUB + sid) * nbatch

    _chunk_loop(sid, lambda c: pltpu.sync_copy(
        hp_hbm.at[pl.ds(c * CH, CH)],
        acc_sh.at[pl.ds(c * CH, CH)]))

    plsc.subcore_barrier()

    pltpu.sync_copy(srcr_hbm.at[pl.ds(base, nbatch)], srcb_v)
    pltpu.sync_copy(dstr_hbm.at[pl.ds(base, nbatch)], dstb_v)
    _edge_phase(nbatch, hp_hbm, acc_sh, srcb_v, dstb_v,
                (b0_v, b1_v), (s0, s1))

    plsc.subcore_barrier()

    _chunk_loop(sid, lambda c: pltpu.sync_copy(
        acc_sh.at[pl.ds(c * CH, CH)],
        out_hbm.at[pl.ds(cid * N + c * CH, CH)]))


def _spmm_l2(hp2p, srcr, dstr):
    hb = E_PAD // (NCORE * NSUB * EB)
    kern = pl.kernel(
        _spmm2_kernel,
        out_type=jax.ShapeDtypeStruct((NCORE * N, DW), jnp.float32),
        mesh=_mesh(),
        scratch_types=[
            pltpu.VMEM((hb, EB), jnp.int32),
            pltpu.VMEM((hb, EB), jnp.int32),
            pltpu.VMEM((EB, DW), jnp.float32),
            pltpu.VMEM((EB, DW), jnp.float32),
            pltpu.VMEM_SHARED((N_ACC, DW), jnp.float32),
            pltpu.SemaphoreType.DMA,
            pltpu.SemaphoreType.DMA,
        ],
    )
    return kern(hp2p, srcr, dstr)


def _dis(d0, d1):
    return lax.rsqrt(d0[:, 0] + d1[:, 0] + 1.0)


def _tc1a_body(x_ref, w_ref, out_ref):
    out_ref[...] = jnp.dot(x_ref[...], w_ref[...],
                           preferred_element_type=jnp.float32)


def _tc1a(x, W1):
    f = pl.pallas_call(
        _tc1a_body,
        grid=(N // NB,),
        in_specs=[
            pl.BlockSpec((NB, DF), lambda i: (i, 0)),
            pl.BlockSpec((DF, DH), lambda i: (0, 0)),
        ],
        out_specs=pl.BlockSpec((NB, DH), lambda i: (i, 0)),
        out_shape=jax.ShapeDtypeStruct((N, DH), jnp.float32),
    )
    return f(x, W1)


def _tc1b_body(h_ref, d0_ref, d1_ref, out_ref):
    dis = _dis(d0_ref[...], d1_ref[...])
    h = h_ref[...]
    out_ref[0] = dis[:, None] * h[:, :DH // 2]
    out_ref[1] = dis[:, None] * h[:, DH // 2:]


def _tc1b(h1, degp):
    f = pl.pallas_call(
        _tc1b_body,
        grid=(N // NB,),
        in_specs=[
            pl.BlockSpec((NB, DH), lambda i: (i, 0)),
            pl.BlockSpec((NB, DW), lambda i: (i, 0)),
            pl.BlockSpec((NB, DW), lambda i: (N // NB + i, 0)),
        ],
        out_specs=pl.BlockSpec((2, NB, DH // 2), lambda i: (0, i, 0)),
        out_shape=jax.ShapeDtypeStruct((2, N, DH // 2), jnp.float32),
    )
    return f(h1, degp, degp).reshape(2 * N, DH // 2)


def _tc2_body(a0_ref, a1_ref, d0_ref, d1_ref, w_ref, b_ref, out_ref):
    dis = _dis(d0_ref[...], d1_ref[...])
    z0 = jax.nn.relu(dis[:, None] * a0_ref[...] + b_ref[0][None, :])
    z1 = jax.nn.relu(dis[:, None] * a1_ref[...] + b_ref[1][None, :])
    h2 = (jnp.dot(z0, w_ref[:DH // 2],
                  preferred_element_type=jnp.float32)
          + jnp.dot(z1, w_ref[DH // 2:],
                    preferred_element_type=jnp.float32))
    out_ref[...] = jnp.concatenate(
        [dis[:, None] * h2, jnp.zeros((NB, DC), jnp.float32)], axis=1)


def _tc2(aggr1, degp, W2, b1):
    f = pl.pallas_call(
        _tc2_body,
        grid=(N // NB,),
        in_specs=[
            pl.BlockSpec((NB, DH // 2), lambda i: (i, 0)),
            pl.BlockSpec((NB, DH // 2), lambda i: (N // NB + i, 0)),
            pl.BlockSpec((NB, DW), lambda i: (i, 0)),
            pl.BlockSpec((NB, DW), lambda i: (N // NB + i, 0)),
            pl.BlockSpec((DH, DC), lambda i: (0, 0)),
            pl.BlockSpec((2, DH // 2), lambda i: (0, 0)),
        ],
        out_specs=pl.BlockSpec((NB, 2 * DC), lambda i: (i, 0)),
        out_shape=jax.ShapeDtypeStruct((N, 2 * DC), jnp.float32),
    )
    return f(aggr1, aggr1, degp, degp, W2, b1.reshape(2, DH // 2))


def _tc3_body(p0_ref, p1_ref, hp_ref, d0_ref, d1_ref, b_ref, out_ref):
    dis = _dis(d0_ref[...], d1_ref[...])
    aggr = (p0_ref[...] + p1_ref[...] - hp_ref[...])[:, :DC]
    logits = dis[:, None] * aggr + b_ref[...]
    m = jnp.max(logits, axis=1, keepdims=True)
    lse = m + jnp.log(jnp.sum(jnp.exp(logits - m), axis=1, keepdims=True))
    out_ref[...] = logits - lse


def _tc3(parts2, hp2p, degp, b2):
    f = pl.pallas_call(
        _tc3_body,
        grid=(N // NB,),
        in_specs=[
            pl.BlockSpec((NB, 2 * DC), lambda i: (i, 0)),
            pl.BlockSpec((NB, 2 * DC), lambda i: (N // NB + i, 0)),
            pl.BlockSpec((NB, 2 * DC), lambda i: (i, 0)),
            pl.BlockSpec((NB, DW), lambda i: (i, 0)),
            pl.BlockSpec((NB, DW), lambda i: (N // NB + i, 0)),
            pl.BlockSpec((1, DC), lambda i: (0, 0)),
        ],
        out_specs=pl.BlockSpec((NB, DC), lambda i: (i, 0)),
        out_shape=jax.ShapeDtypeStruct((N, DC), jnp.float32),
    )
    return f(parts2, parts2, hp2p, degp, degp, b2.reshape(1, DC))


def kernel(x, edge_index, W1, b1, W2, b2):
    src = edge_index[0].astype(jnp.int32)
    dst = edge_index[1].astype(jnp.int32)
    pad = E_PAD - E
    k = jnp.arange(pad, dtype=jnp.int32)
    src_p = jnp.concatenate([src, k * 97 % N])
    dst_p = jnp.concatenate([dst, DUMP + (k % (N_ACC - N))])
    src2r = jnp.concatenate([src_p, src_p + N]).reshape(-1, EB1)
    dstr1 = dst_p.reshape(-1, EB1)
    srcr = src_p.reshape(-1, EB)
    dstr = dst_p.reshape(-1, EB)

    h1 = _tc1a(x, W1)
    degp = _deg_counts(dstr)
    hp1 = _tc1b(h1, degp)
    aggr1 = _spmm(hp1, src2r, dstr1)
    hp2p = _tc2(aggr1, degp, W2, b1)
    parts2 = _spmm_l2(hp2p, srcr, dstr)
    return _tc3(parts2, hp2p, degp, b2)
